# Initial kernel scaffold; baseline (speedup 1.0000x reference)
import jax
import jax.numpy as jnp
from jax.experimental import pallas as pl


def kernel(H, edge_index, W, bias):
    raise NotImplementedError("write your pallas kernel here")


# speedup vs baseline: 1.4971x; 1.4971x over previous
import dataclasses

import jax
import jax.numpy as jnp
from jax import lax
from jax.experimental import pallas as pl
from jax.experimental.pallas import tpu as pltpu
from jax.experimental.pallas import tpu_sc as plsc

N = 10000
N_PAD = 10240
DEN_R = N_PAD // 128
D = 128
E = 320000
NC, NS = 2, 16
EPW = E // (NC * NS)
CHUNK = 80
NCHUNK = EPW // CHUNK
GROUPS = CHUNK // 16
ROWS_PER_TILE = N_PAD // NS


def _mm_body(h_ref, w_ref, o_ref):
    o_ref[...] = jnp.dot(h_ref[...], w_ref[...],
                         preferred_element_type=jnp.float32)


def _matmul(H, W):
    return pl.pallas_call(
        _mm_body,
        out_shape=jax.ShapeDtypeStruct((N, D), jnp.float32),
    )(H, W)


def _edge_body(hp_hbm, src_hbm, dst_hbm, zero_hbm, acc_hbm, dout_hbm,
               sidx, didx, srows, drows, valbuf, den, idv,
               acc_sh, den_sh):
    cid = lax.axis_index("c")
    sid = lax.axis_index("s")

    r0 = sid * ROWS_PER_TILE
    pltpu.sync_copy(zero_hbm.at[pl.ds(r0, ROWS_PER_TILE)],
                    acc_sh.at[pl.ds(r0, ROWS_PER_TILE)])

    @pl.when(sid == 0)
    def _zero_den_sh():
        pltpu.sync_copy(zero_hbm.at[pl.ds(0, DEN_R)], den_sh)

    @pl.loop(0, DEN_R)
    def _zero_den(r):
        @pl.loop(0, 128, step=16)
        def _(c):
            den[r, pl.ds(c, 16)] = jnp.zeros((16,), jnp.float32)

    @pl.loop(0, DEN_R, step=16)
    def _fill_idv(k):
        idv[pl.ds(k, 16)] = lax.iota(jnp.int32, 16) + k

    plsc.subcore_barrier()

    ebase = (cid * NS + sid) * EPW

    @pl.loop(0, NCHUNK)
    def _chunk(c):
        off = ebase + c * CHUNK
        pltpu.sync_copy(src_hbm.at[pl.ds(off, CHUNK)], sidx)
        pltpu.sync_copy(dst_hbm.at[pl.ds(off, CHUNK)], didx)
        pltpu.sync_copy(hp_hbm.at[sidx], srows)
        pltpu.sync_copy(hp_hbm.at[didx], drows)

        @pl.loop(0, GROUPS)
        def _group(g):
            eidx = lax.iota(jnp.int32, 16) + g * 16
            parts = [jnp.zeros((16,), jnp.float32) for _ in range(4)]
            for d in range(D):
                idd = jnp.full((16,), d, jnp.int32)
                a = plsc.load_gather(srows, [eidx, idd])
                b = plsc.load_gather(drows, [eidx, idd])
                parts[d % 4] = parts[d % 4] + a * b
            score = (parts[0] + parts[1]) + (parts[2] + parts[3])
            score = jnp.where(score > 0.0, score, score * 0.2)
            ex = jnp.exp(score)
            for d in range(D):
                idd = jnp.full((16,), d, jnp.int32)
                vd = plsc.load_gather(drows, [eidx, idd])
                plsc.store_scatter(valbuf, [eidx, idd], vd * ex)
            sv = plsc.load_gather(sidx, [eidx])
            ir = lax.shift_right_logical(sv, 7)
            ic = lax.bitwise_and(sv, 127)
            lanes = lax.iota(jnp.int32, 16)
            for l in range(16):
                m = lanes == l
                cur = plsc.load_gather(den, [ir, ic], mask=m)
                plsc.store_scatter(den, [ir, ic], cur + ex, mask=m)

        pltpu.sync_copy(valbuf, acc_sh.at[sidx], add=True)

    pltpu.sync_copy(den, den_sh.at[idv], add=True)

    plsc.subcore_barrier()
    pltpu.sync_copy(acc_sh.at[pl.ds(r0, ROWS_PER_TILE)],
                    acc_hbm.at[cid, pl.ds(r0, ROWS_PER_TILE)])

    @pl.when(sid == 0)
    def _write_den():
        pltpu.sync_copy(den_sh, dout_hbm.at[cid])


def _edge_pass(H_prime, src, dst, zeros):
    mesh = plsc.VectorSubcoreMesh(core_axis_name="c", subcore_axis_name="s")
    cp = pltpu.CompilerParams()
    if "needs_layout_passes" in pltpu.CompilerParams.__dataclass_fields__:
        cp = dataclasses.replace(cp, needs_layout_passes=False)
    kern = pl.kernel(
        _edge_body,
        out_type=(jax.ShapeDtypeStruct((NC, N_PAD, D), jnp.float32),
                  jax.ShapeDtypeStruct((NC, DEN_R, 128), jnp.float32)),
        mesh=mesh,
        scratch_types=[
            pltpu.VMEM((CHUNK,), jnp.int32),
            pltpu.VMEM((CHUNK,), jnp.int32),
            pltpu.VMEM((CHUNK, D), jnp.float32),
            pltpu.VMEM((CHUNK, D), jnp.float32),
            pltpu.VMEM((CHUNK, D), jnp.float32),
            pltpu.VMEM((DEN_R, 128), jnp.float32),
            pltpu.VMEM((DEN_R,), jnp.int32),
            pltpu.VMEM_SHARED((N_PAD, D), jnp.float32),
            pltpu.VMEM_SHARED((DEN_R, 128), jnp.float32),
        ],
        compiler_params=cp,
    )
    return kern(H_prime, src, dst, zeros)


def _combine_body(a_ref, d_ref, b_ref, o_ref):
    num = a_ref[0] + a_ref[1]
    den = d_ref[0] + d_ref[1]
    o_ref[...] = num / (den + 1e-10) + b_ref[...]


def _combine(acc, den, bias):
    return pl.pallas_call(
        _combine_body,
        out_shape=jax.ShapeDtypeStruct((N, D), jnp.float32),
    )(acc, den, bias.reshape(1, D))


def kernel(H, edge_index, W, bias):
    src = edge_index[0].astype(jnp.int32)
    dst = edge_index[1].astype(jnp.int32)
    H_prime = _matmul(H, W)
    zeros = jnp.zeros((N_PAD, D), jnp.float32)
    acc, den = _edge_pass(H_prime, src, dst, zeros)
    acc_n = acc[:, :N, :]
    den_n = den.reshape(NC, N_PAD)[:, :N].reshape(NC, N, 1)
    return _combine(acc_n, den_n, bias)


# speedup vs baseline: 4.1059x; 2.7425x over previous
import dataclasses

import jax
import jax.numpy as jnp
from jax import lax
from jax.experimental import pallas as pl
from jax.experimental.pallas import tpu as pltpu
from jax.experimental.pallas import tpu_sc as plsc

N = 10000
N_PAD = 10240
DEN_R = N_PAD // 128
D = 128
E = 320000
NC, NS = 2, 16
EPW = E // (NC * NS)
CHUNK = 400
NCHUNK = EPW // CHUNK
CHUNK2 = 80
NCHUNK2 = EPW // CHUNK2
GROUPS = CHUNK2 // 16
ROWS_PER_TILE = N_PAD // NS
BE = 512
NBLK = E // BE


def _mm_body(h_ref, w_ref, o_ref):
    o_ref[...] = jnp.dot(h_ref[...], w_ref[...],
                         preferred_element_type=jnp.float32)


def _matmul(H, W):
    return pl.pallas_call(
        _mm_body,
        out_shape=jax.ShapeDtypeStruct((N, D), jnp.float32),
    )(H, W)


def _sc_params():
    cp = pltpu.CompilerParams()
    if "needs_layout_passes" in pltpu.CompilerParams.__dataclass_fields__:
        cp = dataclasses.replace(cp, needs_layout_passes=False)
    return cp


def _gather_body(hp_hbm, src_hbm, dst_hbm, sout_hbm, dout_hbm,
                 sidx, didx, srows, drows):
    cid = lax.axis_index("c")
    sid = lax.axis_index("s")
    ebase = (cid * NS + sid) * EPW

    @pl.loop(0, NCHUNK)
    def _chunk(c):
        off = ebase + c * CHUNK
        pltpu.sync_copy(src_hbm.at[pl.ds(off, CHUNK)], sidx)
        pltpu.sync_copy(dst_hbm.at[pl.ds(off, CHUNK)], didx)
        pltpu.sync_copy(hp_hbm.at[sidx], srows)
        pltpu.sync_copy(srows, sout_hbm.at[pl.ds(off, CHUNK)])
        pltpu.sync_copy(hp_hbm.at[didx], drows)
        pltpu.sync_copy(drows, dout_hbm.at[pl.ds(off, CHUNK)])


def _gather_pass(H_prime, src, dst):
    mesh = plsc.VectorSubcoreMesh(core_axis_name="c", subcore_axis_name="s")
    kern = pl.kernel(
        _gather_body,
        out_type=(jax.ShapeDtypeStruct((E, D), jnp.float32),
                  jax.ShapeDtypeStruct((E, D), jnp.float32)),
        mesh=mesh,
        scratch_types=[
            pltpu.VMEM((CHUNK,), jnp.int32),
            pltpu.VMEM((CHUNK,), jnp.int32),
            pltpu.VMEM((CHUNK, D), jnp.float32),
            pltpu.VMEM((CHUNK, D), jnp.float32),
        ],
        compiler_params=_sc_params(),
    )
    return kern(H_prime, src, dst)


def _score_body(s_ref, d_ref, sc_ref, ex_ref):
    s = s_ref[...]
    d = d_ref[...]
    sc = jnp.sum(s * d, axis=1)
    sc = jnp.where(sc > 0.0, sc, sc * 0.2)
    ex = jnp.exp(sc)
    sc_ref[...] = d * ex[:, None]
    ex_ref[...] = ex


def _score_pass(SRC, DST):
    scaled, ex = pl.pallas_call(
        _score_body,
        grid=(NBLK,),
        in_specs=[
            pl.BlockSpec((BE, D), lambda i: (i, 0)),
            pl.BlockSpec((BE, D), lambda i: (i, 0)),
        ],
        out_specs=[
            pl.BlockSpec((BE, D), lambda i: (i, 0)),
            pl.BlockSpec((BE,), lambda i: (i,)),
        ],
        out_shape=(jax.ShapeDtypeStruct((E, D), jnp.float32),
                   jax.ShapeDtypeStruct((E,), jnp.float32)),
    )(SRC, DST)
    return scaled, ex


def _scatter_body(scaled_hbm, ex_hbm, src_hbm, zero_hbm, acc_hbm, dout_hbm,
                  sidx, rows, exv, den, idv, acc_sh, den_sh):
    cid = lax.axis_index("c")
    sid = lax.axis_index("s")
    r0 = sid * ROWS_PER_TILE

    pltpu.sync_copy(zero_hbm.at[pl.ds(r0, ROWS_PER_TILE)],
                    acc_sh.at[pl.ds(r0, ROWS_PER_TILE)])

    @pl.when(sid == 0)
    def _zero_den_sh():
        pltpu.sync_copy(zero_hbm.at[pl.ds(0, DEN_R)], den_sh)

    zv = jnp.zeros((16,), jnp.float32)

    @pl.loop(0, DEN_R)
    def _zero_den(r):
        for k in range(D // 16):
            den[r, pl.ds(k * 16, 16)] = zv

    @pl.loop(0, DEN_R, step=16)
    def _fill_idv(k):
        idv[pl.ds(k, 16)] = lax.iota(jnp.int32, 16) + k

    plsc.subcore_barrier()

    ebase = (cid * NS + sid) * EPW
    lanes = lax.iota(jnp.int32, 16)

    @pl.loop(0, NCHUNK2)
    def _chunk(c):
        off = ebase + c * CHUNK2
        pltpu.sync_copy(src_hbm.at[pl.ds(off, CHUNK2)], sidx)
        pltpu.sync_copy(scaled_hbm.at[pl.ds(off, CHUNK2)], rows)
        pltpu.sync_copy(ex_hbm.at[pl.ds(off, CHUNK2)], exv)
        pltpu.sync_copy(rows, acc_sh.at[sidx], add=True)

        @pl.loop(0, GROUPS)
        def _group(g):
            ex = plsc.load_gather(exv, [lanes + g * 16])
            sv = plsc.load_gather(sidx, [lanes + g * 16])
            ir = lax.shift_right_logical(sv, 7)
            ic = lax.bitwise_and(sv, 127)
            for l in range(16):
                m = lanes == l
                cur = plsc.load_gather(den, [ir, ic], mask=m)
                plsc.store_scatter(den, [ir, ic], cur + ex, mask=m)

    pltpu.sync_copy(den, den_sh.at[idv], add=True)

    plsc.subcore_barrier()
    pltpu.sync_copy(acc_sh.at[pl.ds(r0, ROWS_PER_TILE)],
                    acc_hbm.at[cid, pl.ds(r0, ROWS_PER_TILE)])

    @pl.when(sid == 0)
    def _write_den():
        pltpu.sync_copy(den_sh, dout_hbm.at[cid])


def _scatter_pass(scaled, ex, src, zeros):
    mesh = plsc.VectorSubcoreMesh(core_axis_name="c", subcore_axis_name="s")
    kern = pl.kernel(
        _scatter_body,
        out_type=(jax.ShapeDtypeStruct((NC, N_PAD, D), jnp.float32),
                  jax.ShapeDtypeStruct((NC, DEN_R, 128), jnp.float32)),
        mesh=mesh,
        scratch_types=[
            pltpu.VMEM((CHUNK2,), jnp.int32),
            pltpu.VMEM((CHUNK2, D), jnp.float32),
            pltpu.VMEM((CHUNK2,), jnp.float32),
            pltpu.VMEM((DEN_R, 128), jnp.float32),
            pltpu.VMEM((DEN_R,), jnp.int32),
            pltpu.VMEM_SHARED((N_PAD, D), jnp.float32),
            pltpu.VMEM_SHARED((DEN_R, 128), jnp.float32),
        ],
        compiler_params=_sc_params(),
    )
    return kern(scaled, ex, src, zeros)


def _combine_body(a_ref, d_ref, b_ref, o_ref):
    num = a_ref[0] + a_ref[1]
    den = d_ref[0] + d_ref[1]
    o_ref[...] = num / (den + 1e-10) + b_ref[...]


def _combine(acc, den, bias):
    return pl.pallas_call(
        _combine_body,
        out_shape=jax.ShapeDtypeStruct((N, D), jnp.float32),
    )(acc, den, bias.reshape(1, D))


def kernel(H, edge_index, W, bias):
    src = edge_index[0].astype(jnp.int32)
    dst = edge_index[1].astype(jnp.int32)
    H_prime = _matmul(H, W)
    SRC, DST = _gather_pass(H_prime, src, dst)
    scaled, ex = _score_pass(SRC, DST)
    zeros = jnp.zeros((N_PAD, D), jnp.float32)
    acc, den = _scatter_pass(scaled, ex, src, zeros)
    acc_n = acc[:, :N, :]
    den_n = den.reshape(NC, N_PAD)[:, :N].reshape(NC, N, 1)
    return _combine(acc_n, den_n, bias)


# speedup vs baseline: 5.4102x; 1.3177x over previous
import dataclasses

import jax
import jax.numpy as jnp
from jax import lax
from jax.experimental import pallas as pl
from jax.experimental.pallas import tpu as pltpu
from jax.experimental.pallas import tpu_sc as plsc

N = 10000
N_PAD = 10240
DEN_R = N_PAD // 128
D = 128
E = 320000
NC, NS = 2, 16
EPW = E // (NC * NS)
CHUNK = 400
NCHUNK = EPW // CHUNK
CHUNK2 = 80
NCHUNK2 = EPW // CHUNK2
GROUPS = CHUNK2 // 16
ROWS_PER_TILE = N_PAD // NS
BE = 4096
NBLK = -(-E // BE)


def _mm_body(h_ref, w_ref, o_ref):
    o_ref[...] = jnp.dot(h_ref[...], w_ref[...],
                         preferred_element_type=jnp.float32)


def _matmul(H, W):
    return pl.pallas_call(
        _mm_body,
        out_shape=jax.ShapeDtypeStruct((N, D), jnp.float32),
    )(H, W)


def _sc_params():
    cp = pltpu.CompilerParams()
    if "needs_layout_passes" in pltpu.CompilerParams.__dataclass_fields__:
        cp = dataclasses.replace(cp, needs_layout_passes=False)
    return cp


def _gather_body(hp_hbm, src_hbm, dst_hbm, sout_hbm, dout_hbm,
                 sidx, didx, srows, drows):
    cid = lax.axis_index("c")
    sid = lax.axis_index("s")
    ebase = (cid * NS + sid) * EPW

    @pl.loop(0, NCHUNK)
    def _chunk(c):
        off = ebase + c * CHUNK
        pltpu.sync_copy(src_hbm.at[pl.ds(off, CHUNK)], sidx)
        pltpu.sync_copy(dst_hbm.at[pl.ds(off, CHUNK)], didx)
        pltpu.sync_copy(hp_hbm.at[sidx], srows)
        pltpu.sync_copy(srows, sout_hbm.at[pl.ds(off, CHUNK)])
        pltpu.sync_copy(hp_hbm.at[didx], drows)
        pltpu.sync_copy(drows, dout_hbm.at[pl.ds(off, CHUNK)])


def _gather_pass(H_prime, src, dst):
    mesh = plsc.VectorSubcoreMesh(core_axis_name="c", subcore_axis_name="s")
    kern = pl.kernel(
        _gather_body,
        out_type=(jax.ShapeDtypeStruct((E, D), jnp.float32),
                  jax.ShapeDtypeStruct((E, D), jnp.float32)),
        mesh=mesh,
        scratch_types=[
            pltpu.VMEM((CHUNK,), jnp.int32),
            pltpu.VMEM((CHUNK,), jnp.int32),
            pltpu.VMEM((CHUNK, D), jnp.float32),
            pltpu.VMEM((CHUNK, D), jnp.float32),
        ],
        compiler_params=_sc_params(),
    )
    return kern(H_prime, src, dst)


def _score_body(s_ref, d_ref, sc_ref, ex_ref):
    s = s_ref[...]
    d = d_ref[...]
    sc = jnp.sum(s * d, axis=1)
    sc = jnp.where(sc > 0.0, sc, sc * 0.2)
    ex = jnp.exp(sc)
    sc_ref[...] = d * ex[:, None]
    ex_ref[...] = ex


def _score_pass(SRC, DST):
    scaled, ex = pl.pallas_call(
        _score_body,
        grid=(NBLK,),
        in_specs=[
            pl.BlockSpec((BE, D), lambda i: (i, 0)),
            pl.BlockSpec((BE, D), lambda i: (i, 0)),
        ],
        out_specs=[
            pl.BlockSpec((BE, D), lambda i: (i, 0)),
            pl.BlockSpec((BE,), lambda i: (i,)),
        ],
        out_shape=(jax.ShapeDtypeStruct((E, D), jnp.float32),
                   jax.ShapeDtypeStruct((E,), jnp.float32)),
    )(SRC, DST)
    return scaled, ex


def _scatter_body(scaled_hbm, ex_hbm, src_hbm, zero_hbm, acc_hbm, dout_hbm,
                  sidx, rows, exv, den, idv, acc_sh, den_sh):
    cid = lax.axis_index("c")
    sid = lax.axis_index("s")
    r0 = sid * ROWS_PER_TILE

    pltpu.sync_copy(zero_hbm.at[pl.ds(r0, ROWS_PER_TILE)],
                    acc_sh.at[pl.ds(r0, ROWS_PER_TILE)])

    @pl.when(sid == 0)
    def _zero_den_sh():
        pltpu.sync_copy(zero_hbm.at[pl.ds(0, DEN_R)], den_sh)

    zv = jnp.zeros((16,), jnp.float32)

    @pl.loop(0, DEN_R)
    def _zero_den(r):
        for k in range(D // 16):
            den[r, pl.ds(k * 16, 16)] = zv

    @pl.loop(0, DEN_R, step=16)
    def _fill_idv(k):
        idv[pl.ds(k, 16)] = lax.iota(jnp.int32, 16) + k

    plsc.subcore_barrier()

    ebase = (cid * NS + sid) * EPW
    lanes = lax.iota(jnp.int32, 16)

    @pl.loop(0, NCHUNK2)
    def _chunk(c):
        off = ebase + c * CHUNK2
        pltpu.sync_copy(src_hbm.at[pl.ds(off, CHUNK2)], sidx)
        pltpu.sync_copy(scaled_hbm.at[pl.ds(off, CHUNK2)], rows)
        pltpu.sync_copy(ex_hbm.at[pl.ds(off, CHUNK2)], exv)
        pltpu.sync_copy(rows, acc_sh.at[sidx], add=True)

        @pl.loop(0, GROUPS)
        def _group(g):
            ex = plsc.load_gather(exv, [lanes + g * 16])
            sv = plsc.load_gather(sidx, [lanes + g * 16])
            ir = lax.shift_right_logical(sv, 7)
            ic = lax.bitwise_and(sv, 127)
            for l in range(16):
                m = lanes == l
                cur = plsc.load_gather(den, [ir, ic], mask=m)
                plsc.store_scatter(den, [ir, ic], cur + ex, mask=m)

    pltpu.sync_copy(den, den_sh.at[idv], add=True)

    plsc.subcore_barrier()
    pltpu.sync_copy(acc_sh.at[pl.ds(r0, ROWS_PER_TILE)],
                    acc_hbm.at[cid, pl.ds(r0, ROWS_PER_TILE)])

    @pl.when(sid == 0)
    def _write_den():
        pltpu.sync_copy(den_sh, dout_hbm.at[cid])


def _scatter_pass(scaled, ex, src, zeros):
    mesh = plsc.VectorSubcoreMesh(core_axis_name="c", subcore_axis_name="s")
    kern = pl.kernel(
        _scatter_body,
        out_type=(jax.ShapeDtypeStruct((NC, N_PAD, D), jnp.float32),
                  jax.ShapeDtypeStruct((NC, DEN_R, 128), jnp.float32)),
        mesh=mesh,
        scratch_types=[
            pltpu.VMEM((CHUNK2,), jnp.int32),
            pltpu.VMEM((CHUNK2, D), jnp.float32),
            pltpu.VMEM((CHUNK2,), jnp.float32),
            pltpu.VMEM((DEN_R, 128), jnp.float32),
            pltpu.VMEM((DEN_R,), jnp.int32),
            pltpu.VMEM_SHARED((N_PAD, D), jnp.float32),
            pltpu.VMEM_SHARED((DEN_R, 128), jnp.float32),
        ],
        compiler_params=_sc_params(),
    )
    return kern(scaled, ex, src, zeros)


def _combine_body(a_ref, d_ref, b_ref, o_ref):
    num = a_ref[0] + a_ref[1]
    den = d_ref[0] + d_ref[1]
    o_ref[...] = num / (den + 1e-10) + b_ref[...]


def _combine(acc, den, bias):
    return pl.pallas_call(
        _combine_body,
        out_shape=jax.ShapeDtypeStruct((N, D), jnp.float32),
    )(acc, den, bias.reshape(1, D))


def kernel(H, edge_index, W, bias):
    src = edge_index[0].astype(jnp.int32)
    dst = edge_index[1].astype(jnp.int32)
    H_prime = _matmul(H, W)
    SRC, DST = _gather_pass(H_prime, src, dst)
    scaled, ex = _score_pass(SRC, DST)
    zeros = jnp.zeros((N_PAD, D), jnp.float32)
    acc, den = _scatter_pass(scaled, ex, src, zeros)
    acc_n = acc[:, :N, :]
    den_n = den.reshape(NC, N_PAD)[:, :N].reshape(NC, N, 1)
    return _combine(acc_n, den_n, bias)


# speedup vs baseline: 6.2892x; 1.1625x over previous
import dataclasses

import jax
import jax.numpy as jnp
from jax import lax
from jax.experimental import pallas as pl
from jax.experimental.pallas import tpu as pltpu
from jax.experimental.pallas import tpu_sc as plsc

N = 10000
N_PAD = 10240
DEN_R = N_PAD // 128
D = 128
E = 320000
NC, NS = 2, 16
EPW = E // (NC * NS)
CHUNK = 400
NCHUNK = EPW // CHUNK
CHUNK2 = 80
NCHUNK2 = EPW // CHUNK2
GROUPS = CHUNK2 // 16
ROWS_PER_TILE = N_PAD // NS
BE = 4096
NBLK = -(-E // BE)


def _mm_body(h_ref, w_ref, o_ref):
    o_ref[...] = jnp.dot(h_ref[...], w_ref[...],
                         preferred_element_type=jnp.float32)


def _matmul(H, W):
    return pl.pallas_call(
        _mm_body,
        out_shape=jax.ShapeDtypeStruct((N, D), jnp.float32),
    )(H, W)


def _sc_params():
    cp = pltpu.CompilerParams()
    if "needs_layout_passes" in pltpu.CompilerParams.__dataclass_fields__:
        cp = dataclasses.replace(cp, needs_layout_passes=False)
    return cp


def _gather_body(hp_hbm, src_hbm, dst_hbm, sout_hbm, dout_hbm,
                 sidx, didx, srows, drows):
    cid = lax.axis_index("c")
    sid = lax.axis_index("s")
    wid = cid * NS + sid
    ebase = wid * EPW

    pltpu.sync_copy(src_hbm.at[pl.ds(ebase, EPW)], sidx)
    pltpu.sync_copy(dst_hbm.at[pl.ds(ebase, EPW)], didx)

    @pl.loop(0, NCHUNK)
    def _chunk(c):
        off = ebase + c * CHUNK
        pltpu.sync_copy(hp_hbm.at[sidx.at[pl.ds(c * CHUNK, CHUNK)]], srows)
        pltpu.sync_copy(srows, sout_hbm.at[pl.ds(off, CHUNK)])
        pltpu.sync_copy(hp_hbm.at[didx.at[pl.ds(c * CHUNK, CHUNK)]], drows)
        pltpu.sync_copy(drows, dout_hbm.at[pl.ds(off, CHUNK)])


def _gather_pass(H_prime, src, dst):
    mesh = plsc.VectorSubcoreMesh(core_axis_name="c", subcore_axis_name="s")
    kern = pl.kernel(
        _gather_body,
        out_type=(jax.ShapeDtypeStruct((E, D), jnp.float32),
                  jax.ShapeDtypeStruct((E, D), jnp.float32)),
        mesh=mesh,
        scratch_types=[
            pltpu.VMEM((EPW,), jnp.int32),
            pltpu.VMEM((EPW,), jnp.int32),
            pltpu.VMEM((CHUNK, D), jnp.float32),
            pltpu.VMEM((CHUNK, D), jnp.float32),
        ],
        compiler_params=_sc_params(),
    )
    return kern(H_prime, src, dst)


def _score_body(s_ref, d_ref, sc_ref, ex_ref):
    s = s_ref[...]
    d = d_ref[...]
    sc = jnp.sum(s * d, axis=1)
    sc = jnp.where(sc > 0.0, sc, sc * 0.2)
    ex = jnp.exp(sc)
    sc_ref[...] = d * ex[:, None]
    ex_ref[...] = ex


def _score_pass(SRC, DST):
    scaled, ex = pl.pallas_call(
        _score_body,
        grid=(NBLK,),
        in_specs=[
            pl.BlockSpec((BE, D), lambda i: (i, 0)),
            pl.BlockSpec((BE, D), lambda i: (i, 0)),
        ],
        out_specs=[
            pl.BlockSpec((BE, D), lambda i: (i, 0)),
            pl.BlockSpec((BE,), lambda i: (i,)),
        ],
        out_shape=(jax.ShapeDtypeStruct((E, D), jnp.float32),
                   jax.ShapeDtypeStruct((E,), jnp.float32)),
    )(SRC, DST)
    return scaled, ex


def _scatter_body(scaled_hbm, ex_hbm, src_hbm, zero_hbm, acc_hbm, dout_hbm,
                  sidx, sidx_c, rows, exv, den, idv, acc_sh, den_sh):
    cid = lax.axis_index("c")
    sid = lax.axis_index("s")
    r0 = sid * ROWS_PER_TILE

    pltpu.sync_copy(zero_hbm.at[pl.ds(r0, ROWS_PER_TILE)],
                    acc_sh.at[pl.ds(r0, ROWS_PER_TILE)])

    @pl.when(sid == 0)
    def _zero_den_sh():
        pltpu.sync_copy(zero_hbm.at[pl.ds(0, DEN_R)], den_sh)

    zv = jnp.zeros((16,), jnp.float32)

    @pl.loop(0, DEN_R)
    def _zero_den(r):
        for k in range(D // 16):
            den[r, pl.ds(k * 16, 16)] = zv

    @pl.loop(0, DEN_R, step=16)
    def _fill_idv(k):
        idv[pl.ds(k, 16)] = lax.iota(jnp.int32, 16) + k

    plsc.subcore_barrier()

    wid = cid * NS + sid
    ebase = wid * EPW
    lanes = lax.iota(jnp.int32, 16)

    pltpu.sync_copy(src_hbm.at[pl.ds(ebase, EPW)], sidx)
    pltpu.sync_copy(ex_hbm.at[pl.ds(ebase, EPW)], exv)

    @pl.loop(0, NCHUNK2)
    def _chunk(c):
        off = ebase + c * CHUNK2
        pltpu.sync_copy(scaled_hbm.at[pl.ds(off, CHUNK2)], rows)
        for k in range(GROUPS):
            sidx_c[pl.ds(k * 16, 16)] = plsc.load_gather(
                sidx, [lanes + c * CHUNK2 + k * 16])
        pltpu.sync_copy(rows, acc_sh.at[sidx_c], add=True)

        @pl.loop(0, GROUPS)
        def _group(g):
            ex = plsc.load_gather(exv, [lanes + c * CHUNK2 + g * 16])
            sv = plsc.load_gather(sidx_c, [lanes + g * 16])
            ir = lax.shift_right_logical(sv, 7)
            ic = lax.bitwise_and(sv, 127)
            for l in range(16):
                m = lanes == l
                cur = plsc.load_gather(den, [ir, ic], mask=m)
                plsc.store_scatter(den, [ir, ic], cur + ex, mask=m)

    pltpu.sync_copy(den, den_sh.at[idv], add=True)

    plsc.subcore_barrier()
    pltpu.sync_copy(acc_sh.at[pl.ds(r0, ROWS_PER_TILE)],
                    acc_hbm.at[cid, pl.ds(r0, ROWS_PER_TILE)])

    @pl.when(sid == 0)
    def _write_den():
        pltpu.sync_copy(den_sh, dout_hbm.at[cid])


def _scatter_pass(scaled, ex, src, zeros):
    mesh = plsc.VectorSubcoreMesh(core_axis_name="c", subcore_axis_name="s")
    kern = pl.kernel(
        _scatter_body,
        out_type=(jax.ShapeDtypeStruct((NC, N_PAD, D), jnp.float32),
                  jax.ShapeDtypeStruct((NC, DEN_R, 128), jnp.float32)),
        mesh=mesh,
        scratch_types=[
            pltpu.VMEM((EPW,), jnp.int32),
            pltpu.VMEM((CHUNK2,), jnp.int32),
            pltpu.VMEM((CHUNK2, D), jnp.float32),
            pltpu.VMEM((EPW,), jnp.float32),
            pltpu.VMEM((DEN_R, 128), jnp.float32),
            pltpu.VMEM((DEN_R,), jnp.int32),
            pltpu.VMEM_SHARED((N_PAD, D), jnp.float32),
            pltpu.VMEM_SHARED((DEN_R, 128), jnp.float32),
        ],
        compiler_params=_sc_params(),
    )
    return kern(scaled, ex, src, zeros)


def _combine_body(a_ref, d_ref, b_ref, o_ref):
    num = a_ref[0] + a_ref[1]
    den = d_ref[0] + d_ref[1]
    o_ref[...] = num / (den + 1e-10) + b_ref[...]


def _combine(acc, den, bias):
    return pl.pallas_call(
        _combine_body,
        out_shape=jax.ShapeDtypeStruct((N, D), jnp.float32),
    )(acc, den, bias.reshape(1, D))


def kernel(H, edge_index, W, bias):
    src = edge_index[0].astype(jnp.int32)
    dst = edge_index[1].astype(jnp.int32)
    H_prime = _matmul(H, W)
    SRC, DST = _gather_pass(H_prime, src, dst)
    scaled, ex = _score_pass(SRC, DST)
    zeros = jnp.zeros((N_PAD, D), jnp.float32)
    acc, den = _scatter_pass(scaled, ex, src, zeros)
    acc_n = acc[:, :N, :]
    den_n = den.reshape(NC, N_PAD)[:, :N].reshape(NC, N, 1)
    return _combine(acc_n, den_n, bias)


# speedup vs baseline: 6.8970x; 1.0966x over previous
import dataclasses

import jax
import jax.numpy as jnp
from jax import lax
from jax.experimental import pallas as pl
from jax.experimental.pallas import tpu as pltpu
from jax.experimental.pallas import tpu_sc as plsc

N = 10000
N_PAD = 10240
DEN_R = N_PAD // 128
D = 128
E = 320000
NC, NS = 2, 16
EPW = E // (NC * NS)
CHUNK = 400
NCHUNK = EPW // CHUNK
CHUNK2 = 80
NCHUNK2 = EPW // CHUNK2
GROUPS = CHUNK2 // 16
ROWS_PER_TILE = N_PAD // NS
BE = 4096
NBLK = -(-E // BE)


def _mm_body(h_ref, w_ref, o_ref):
    o_ref[...] = jnp.dot(h_ref[...], w_ref[...],
                         preferred_element_type=jnp.float32)


def _matmul(H, W):
    return pl.pallas_call(
        _mm_body,
        out_shape=jax.ShapeDtypeStruct((N, D), jnp.float32),
    )(H, W)


def _sc_params():
    cp = pltpu.CompilerParams()
    if "needs_layout_passes" in pltpu.CompilerParams.__dataclass_fields__:
        cp = dataclasses.replace(cp, needs_layout_passes=False)
    return cp


def _gather_body(hp_hbm, src_hbm, dst_hbm, sout_hbm, dout_hbm,
                 sidx, didx, srows, drows, sem_g, sem_w):
    cid = lax.axis_index("c")
    sid = lax.axis_index("s")
    wid = cid * NS + sid
    ebase = wid * EPW

    pltpu.sync_copy(src_hbm.at[pl.ds(ebase, EPW)], sidx)
    pltpu.sync_copy(dst_hbm.at[pl.ds(ebase, EPW)], didx)

    @pl.loop(0, NCHUNK)
    def _chunk(c):
        off = ebase + c * CHUNK
        ga = pltpu.async_copy(
            hp_hbm.at[sidx.at[pl.ds(c * CHUNK, CHUNK)]], srows, sem_g)
        gb = pltpu.async_copy(
            hp_hbm.at[didx.at[pl.ds(c * CHUNK, CHUNK)]], drows, sem_g)
        ga.wait()
        gb.wait()
        wa = pltpu.async_copy(srows, sout_hbm.at[pl.ds(off, CHUNK)], sem_w)
        wb = pltpu.async_copy(drows, dout_hbm.at[pl.ds(off, CHUNK)], sem_w)
        wa.wait()
        wb.wait()


def _gather_pass(H_prime, src, dst):
    mesh = plsc.VectorSubcoreMesh(core_axis_name="c", subcore_axis_name="s")
    kern = pl.kernel(
        _gather_body,
        out_type=(jax.ShapeDtypeStruct((E, D), jnp.float32),
                  jax.ShapeDtypeStruct((E, D), jnp.float32)),
        mesh=mesh,
        scratch_types=[
            pltpu.VMEM((EPW,), jnp.int32),
            pltpu.VMEM((EPW,), jnp.int32),
            pltpu.VMEM((CHUNK, D), jnp.float32),
            pltpu.VMEM((CHUNK, D), jnp.float32),
            pltpu.SemaphoreType.DMA,
            pltpu.SemaphoreType.DMA,
        ],
        compiler_params=_sc_params(),
    )
    return kern(H_prime, src, dst)


def _score_body(s_ref, d_ref, sc_ref, ex_ref):
    s = s_ref[...]
    d = d_ref[...]
    sc = jnp.sum(s * d, axis=1)
    sc = jnp.where(sc > 0.0, sc, sc * 0.2)
    ex = jnp.exp(sc)
    sc_ref[...] = d * ex[:, None]
    ex_ref[...] = ex


def _score_pass(SRC, DST):
    scaled, ex = pl.pallas_call(
        _score_body,
        grid=(NBLK,),
        in_specs=[
            pl.BlockSpec((BE, D), lambda i: (i, 0)),
            pl.BlockSpec((BE, D), lambda i: (i, 0)),
        ],
        out_specs=[
            pl.BlockSpec((BE, D), lambda i: (i, 0)),
            pl.BlockSpec((BE,), lambda i: (i,)),
        ],
        out_shape=(jax.ShapeDtypeStruct((E, D), jnp.float32),
                   jax.ShapeDtypeStruct((E,), jnp.float32)),
    )(SRC, DST)
    return scaled, ex


def _scatter_body(scaled_hbm, ex_hbm, src_hbm, zero_hbm, acc_hbm, dout_hbm,
                  sidx, sidx_c, rows, exv, den, idv, acc_sh, den_sh, sem_r):
    cid = lax.axis_index("c")
    sid = lax.axis_index("s")
    r0 = sid * ROWS_PER_TILE

    pltpu.sync_copy(zero_hbm.at[pl.ds(r0, ROWS_PER_TILE)],
                    acc_sh.at[pl.ds(r0, ROWS_PER_TILE)])

    @pl.when(sid == 0)
    def _zero_den_sh():
        pltpu.sync_copy(zero_hbm.at[pl.ds(0, DEN_R)], den_sh)

    zv = jnp.zeros((16,), jnp.float32)

    @pl.loop(0, DEN_R)
    def _zero_den(r):
        for k in range(D // 16):
            den[r, pl.ds(k * 16, 16)] = zv

    @pl.loop(0, DEN_R, step=16)
    def _fill_idv(k):
        idv[pl.ds(k, 16)] = lax.iota(jnp.int32, 16) + k

    plsc.subcore_barrier()

    wid = cid * NS + sid
    ebase = wid * EPW
    lanes = lax.iota(jnp.int32, 16)

    pltpu.sync_copy(src_hbm.at[pl.ds(ebase, EPW)], sidx)
    pltpu.sync_copy(ex_hbm.at[pl.ds(ebase, EPW)], exv)

    @pl.loop(0, NCHUNK2)
    def _chunk(c):
        off = ebase + c * CHUNK2
        rd = pltpu.async_copy(scaled_hbm.at[pl.ds(off, CHUNK2)], rows,
                              sem_r)
        for k in range(GROUPS):
            sidx_c[pl.ds(k * 16, 16)] = plsc.load_gather(
                sidx, [lanes + c * CHUNK2 + k * 16])

        @pl.loop(0, GROUPS)
        def _group(g):
            ex = plsc.load_gather(exv, [lanes + c * CHUNK2 + g * 16])
            sv = plsc.load_gather(sidx_c, [lanes + g * 16])
            ir = lax.shift_right_logical(sv, 7)
            ic = lax.bitwise_and(sv, 127)
            for l in range(16):
                m = lanes == l
                cur = plsc.load_gather(den, [ir, ic], mask=m)
                plsc.store_scatter(den, [ir, ic], cur + ex, mask=m)

        rd.wait()
        pltpu.sync_copy(rows, acc_sh.at[sidx_c], add=True)

    pltpu.sync_copy(den, den_sh.at[idv], add=True)

    plsc.subcore_barrier()
    pltpu.sync_copy(acc_sh.at[pl.ds(r0, ROWS_PER_TILE)],
                    acc_hbm.at[cid, pl.ds(r0, ROWS_PER_TILE)])

    @pl.when(sid == 0)
    def _write_den():
        pltpu.sync_copy(den_sh, dout_hbm.at[cid])


def _scatter_pass(scaled, ex, src, zeros):
    mesh = plsc.VectorSubcoreMesh(core_axis_name="c", subcore_axis_name="s")
    kern = pl.kernel(
        _scatter_body,
        out_type=(jax.ShapeDtypeStruct((NC, N_PAD, D), jnp.float32),
                  jax.ShapeDtypeStruct((NC, DEN_R, 128), jnp.float32)),
        mesh=mesh,
        scratch_types=[
            pltpu.VMEM((EPW,), jnp.int32),
            pltpu.VMEM((CHUNK2,), jnp.int32),
            pltpu.VMEM((CHUNK2, D), jnp.float32),
            pltpu.VMEM((EPW,), jnp.float32),
            pltpu.VMEM((DEN_R, 128), jnp.float32),
            pltpu.VMEM((DEN_R,), jnp.int32),
            pltpu.VMEM_SHARED((N_PAD, D), jnp.float32),
            pltpu.VMEM_SHARED((DEN_R, 128), jnp.float32),
            pltpu.SemaphoreType.DMA,
        ],
        compiler_params=_sc_params(),
    )
    return kern(scaled, ex, src, zeros)


def _combine_body(a_ref, d_ref, b_ref, o_ref):
    num = a_ref[0] + a_ref[1]
    den = d_ref[0] + d_ref[1]
    o_ref[...] = num / (den + 1e-10) + b_ref[...]


def _combine(acc, den, bias):
    return pl.pallas_call(
        _combine_body,
        out_shape=jax.ShapeDtypeStruct((N, D), jnp.float32),
    )(acc, den, bias.reshape(1, D))


def kernel(H, edge_index, W, bias):
    src = edge_index[0].astype(jnp.int32)
    dst = edge_index[1].astype(jnp.int32)
    H_prime = _matmul(H, W)
    SRC, DST = _gather_pass(H_prime, src, dst)
    scaled, ex = _score_pass(SRC, DST)
    zeros = jnp.zeros((N_PAD, D), jnp.float32)
    acc, den = _scatter_pass(scaled, ex, src, zeros)
    acc_n = acc[:, :N, :]
    den_n = den.reshape(NC, N_PAD)[:, :N].reshape(NC, N, 1)
    return _combine(acc_n, den_n, bias)
